# baseline (speedup 1.0000x reference)
import jax
import jax.numpy as jnp
import numpy as np
from jax.experimental import pallas as pl
from jax.experimental.pallas import tpu as pltpu

POOL = 7


def _conv2d(x, W, b, pad):
    out = jax.lax.conv_general_dilated(
        x, W, (1, 1), [(pad, pad), (pad, pad)],
        dimension_numbers=('NCHW', 'OIHW', 'NCHW'))
    return out + b[None, :, None, None]


def _upsample_add(x, y):
    B, C, H, W = y.shape
    return jax.image.resize(x, (x.shape[0], x.shape[1], H, W), method='bilinear') + y


def _roi_align(feat, rois, scale, pooled=POOL):
    B, C, H, W = feat.shape
    R = rois.shape[0]
    bi = rois[:, 0].astype(jnp.int32)
    x1 = rois[:, 1] * scale
    y1 = rois[:, 2] * scale
    x2 = rois[:, 3] * scale
    y2 = rois[:, 4] * scale
    bw = jnp.maximum(x2 - x1 + 1.0, 1.0) / pooled
    bh = jnp.maximum(y2 - y1 + 1.0, 1.0) / pooled
    jj = jnp.arange(pooled, dtype=jnp.float32) + 0.5
    xs = x1[:, None] + jj[None, :] * bw[:, None]
    ys = y1[:, None] + jj[None, :] * bh[:, None]
    gx = jnp.broadcast_to(xs[:, None, :], (R, pooled, pooled))
    gy = jnp.broadcast_to(ys[:, :, None], (R, pooled, pooled))
    x0f = jnp.floor(gx)
    y0f = jnp.floor(gy)
    lx = gx - x0f
    ly = gy - y0f
    x0 = jnp.clip(x0f.astype(jnp.int32), 0, W - 1)
    x1i = jnp.clip(x0 + 1, 0, W - 1)
    y0 = jnp.clip(y0f.astype(jnp.int32), 0, H - 1)
    y1i = jnp.clip(y0 + 1, 0, H - 1)
    b3 = bi[:, None, None]
    v00 = feat[b3, :, y0, x0]
    v01 = feat[b3, :, y0, x1i]
    v10 = feat[b3, :, y1i, x0]
    v11 = feat[b3, :, y1i, x1i]
    w00 = ((1.0 - ly) * (1.0 - lx))[..., None]
    w01 = ((1.0 - ly) * lx)[..., None]
    w10 = (ly * (1.0 - lx))[..., None]
    w11 = (ly * lx)[..., None]
    out = v00 * w00 + v01 * w01 + v10 * w10 + v11 * w11
    return jnp.transpose(out, (0, 3, 1, 2))


def _combine_body(lvl_ref, f2_ref, f3_ref, f4_ref, f5_ref, out_ref):
    lvl = lvl_ref[...]
    out = jnp.where(lvl == 2.0, f2_ref[...], 0.0)
    out = jnp.where(lvl == 3.0, f3_ref[...], out)
    out = jnp.where(lvl == 4.0, f4_ref[...], out)
    out = jnp.where(lvl == 5.0, f5_ref[...], out)
    out_ref[...] = out


def kernel(c2, c3, c4, c5, rois, im_info, Wt, bt, Wl1, bl1, Wl2, bl2, Wl3, bl3,
           Ws1, bs1, Ws2, bs2, Ws3, bs3):
    p5 = _conv2d(c5, Wt, bt, 0)
    p4 = _conv2d(_upsample_add(p5, _conv2d(c4, Wl1, bl1, 0)), Ws1, bs1, 1)
    p3 = _conv2d(_upsample_add(p4, _conv2d(c3, Wl2, bl2, 0)), Ws2, bs2, 1)
    p2 = _conv2d(_upsample_add(p3, _conv2d(c2, Wl3, bl3, 0)), Ws3, bs3, 1)
    feats = [p2, p3, p4, p5]

    R = rois.shape[0]
    C = Wt.shape[0]
    h = rois[:, 4] - rois[:, 2] + 1.0
    w = rois[:, 3] - rois[:, 1] + 1.0
    lvl = jnp.floor(jnp.log(jnp.sqrt(h * w) / 224.0) / np.log(2.0) + 4.0)
    lvl = jnp.clip(lvl, 2.0, 5.0)

    fs = []
    for i in range(4):
        scale = feats[i].shape[2] / im_info[0, 0]
        f = _roi_align(feats[i], rois, scale)
        fs.append(f.reshape(R, C * POOL * POOL))

    lvl2d = jnp.broadcast_to(lvl[:, None], (R, 1))
    BLK = 32
    D = C * POOL * POOL
    out = pl.pallas_call(
        _combine_body,
        grid=(R // BLK,),
        in_specs=[pl.BlockSpec((BLK, 1), lambda i: (i, 0))] +
                 [pl.BlockSpec((BLK, D), lambda i: (i, 0)) for _ in range(4)],
        out_specs=pl.BlockSpec((BLK, D), lambda i: (i, 0)),
        out_shape=jax.ShapeDtypeStruct((R, D), jnp.float32),
    )(lvl2d, *fs)
    return out.reshape(R, C, POOL, POOL)


# speedup vs baseline: 3.2295x; 3.2295x over previous
import functools

import jax
import jax.numpy as jnp
import numpy as np
from jax import lax
from jax.experimental import pallas as pl
from jax.experimental.pallas import tpu as pltpu
from jax.experimental.pallas import tpu_sc as plsc

POOL = 7
NPTS = POOL * POOL
NPAD = 56
LEVEL_W = (128, 64, 32, 16)
LEVEL_OFF = (0, 16384, 20480, 21504)
TABLE_ROWS = 21760


def _conv2d(x, W, b, pad):
    out = jax.lax.conv_general_dilated(
        x, W, (1, 1), [(pad, pad), (pad, pad)],
        dimension_numbers=('NCHW', 'OIHW', 'NCHW'))
    return out + b[None, :, None, None]


def _upsample_add(x, y):
    B, C, H, W = y.shape
    return jax.image.resize(x, (x.shape[0], x.shape[1], H, W), method='bilinear') + y


def _roi_meta(rois, im_info):
    R = rois.shape[0]
    h = rois[:, 4] - rois[:, 2] + 1.0
    w = rois[:, 3] - rois[:, 1] + 1.0
    lvl = jnp.floor(jnp.log(jnp.sqrt(h * w) / 224.0) / np.log(2.0) + 4.0)
    lvl = jnp.clip(lvl, 2.0, 5.0)
    li = lvl.astype(jnp.int32) - 2
    Wf = jnp.array(LEVEL_W, jnp.float32)[li]
    Wi = jnp.array(LEVEL_W, jnp.int32)[li]
    off = jnp.array(LEVEL_OFF, jnp.int32)[li]
    scale = Wf / im_info[0, 0]
    x1 = rois[:, 1] * scale
    y1 = rois[:, 2] * scale
    x2 = rois[:, 3] * scale
    y2 = rois[:, 4] * scale
    bw = jnp.maximum(x2 - x1 + 1.0, 1.0) / POOL
    bh = jnp.maximum(y2 - y1 + 1.0, 1.0) / POOL
    jj = jnp.arange(POOL, dtype=jnp.float32) + 0.5
    xs = x1[:, None] + jj[None, :] * bw[:, None]
    ys = y1[:, None] + jj[None, :] * bh[:, None]
    x0f = jnp.floor(xs)
    y0f = jnp.floor(ys)
    lx = xs - x0f
    ly = ys - y0f
    wmax = Wi[:, None] - 1
    x0 = jnp.clip(x0f.astype(jnp.int32), 0, wmax)
    x1i = jnp.clip(x0 + 1, 0, wmax)
    y0 = jnp.clip(y0f.astype(jnp.int32), 0, wmax)
    y1i = jnp.clip(y0 + 1, 0, wmax)
    ry0 = off[:, None] + y0 * Wi[:, None]
    ry1 = off[:, None] + y1i * Wi[:, None]

    def mk(rowy, xcol):
        return (rowy[:, :, None] + xcol[:, None, :]).reshape(R, NPTS)

    def mw(a, b):
        return (a[:, :, None] * b[:, None, :]).reshape(R, NPTS)

    idx = jnp.stack([mk(ry0, x0), mk(ry0, x1i), mk(ry1, x0), mk(ry1, x1i)], 1)
    wts = jnp.stack([mw(1.0 - ly, 1.0 - lx), mw(1.0 - ly, lx),
                     mw(ly, 1.0 - lx), mw(ly, lx)], 1)
    pad = ((0, 0), (0, 0), (0, NPAD - NPTS))
    return jnp.pad(idx, pad).astype(jnp.int32), jnp.pad(wts, pad)


def _sc_gather(table, idx):
    R = idx.shape[0]
    rpw = R // 32
    mesh = plsc.VectorSubcoreMesh(core_axis_name="c", subcore_axis_name="s")

    @functools.partial(
        pl.kernel, mesh=mesh,
        out_type=jax.ShapeDtypeStruct((R, 4, NPAD, 256), jnp.float32),
        scratch_types=[
            pltpu.VMEM((4, NPAD), jnp.int32),
            pltpu.VMEM((4, NPAD, 256), jnp.float32),
            pltpu.SemaphoreType.DMA,
        ],
    )
    def k(table_hbm, idx_hbm, out_hbm, idx_v, rows_v, sem):
        wid = lax.axis_index("s") * 2 + lax.axis_index("c")

        def body(i, carry):
            r = wid * rpw + i
            pltpu.sync_copy(idx_hbm.at[r], idx_v)
            for kk in range(4):
                pltpu.async_copy(table_hbm.at[idx_v.at[kk]],
                                 rows_v.at[kk], sem).wait()
            pltpu.sync_copy(rows_v, out_hbm.at[r])
            return carry

        lax.fori_loop(0, rpw, body, 0)

    return k(table, idx)


def _combine_body(w_ref, g_ref, o_ref):
    acc = (g_ref[:, 0] * w_ref[:, 0] + g_ref[:, 1] * w_ref[:, 1]
           + g_ref[:, 2] * w_ref[:, 2] + g_ref[:, 3] * w_ref[:, 3])
    o_ref[...] = acc[:, :NPTS, :]


def _tc_combine(g, wts):
    R = g.shape[0]
    BR = 8
    return pl.pallas_call(
        _combine_body,
        grid=(R // BR,),
        in_specs=[pl.BlockSpec((BR, 4, NPAD, 1), lambda i: (i, 0, 0, 0)),
                  pl.BlockSpec((BR, 4, NPAD, 256), lambda i: (i, 0, 0, 0))],
        out_specs=pl.BlockSpec((BR, NPTS, 256), lambda i: (i, 0, 0)),
        out_shape=jax.ShapeDtypeStruct((R, NPTS, 256), jnp.float32),
    )(wts[..., None], g)


def kernel(c2, c3, c4, c5, rois, im_info, Wt, bt, Wl1, bl1, Wl2, bl2, Wl3, bl3,
           Ws1, bs1, Ws2, bs2, Ws3, bs3):
    p5 = _conv2d(c5, Wt, bt, 0)
    p4 = _conv2d(_upsample_add(p5, _conv2d(c4, Wl1, bl1, 0)), Ws1, bs1, 1)
    p3 = _conv2d(_upsample_add(p4, _conv2d(c3, Wl2, bl2, 0)), Ws2, bs2, 1)
    p2 = _conv2d(_upsample_add(p3, _conv2d(c2, Wl3, bl3, 0)), Ws3, bs3, 1)

    C = Wt.shape[0]
    table = jnp.concatenate(
        [jnp.transpose(p, (0, 2, 3, 1)).reshape(-1, C) for p in (p2, p3, p4, p5)], 0)

    R = rois.shape[0]
    idx, wts = _roi_meta(rois, im_info)
    g = _sc_gather(table, idx)
    out = _tc_combine(g, wts)
    return jnp.transpose(out.reshape(R, POOL, POOL, C), (0, 3, 1, 2))
